# Initial kernel scaffold; baseline (speedup 1.0000x reference)
import jax
import jax.numpy as jnp
from jax.experimental import pallas as pl


def kernel(X, idx_weight):
    raise NotImplementedError("write your pallas kernel here")


# speedup vs baseline: 292.5707x; 292.5707x over previous
import functools

import jax
import jax.numpy as jnp
from jax import lax
from jax.experimental import pallas as pl
from jax.experimental.pallas import tpu as pltpu
from jax.experimental.pallas import tpu_sc as plsc

_NUM_ROWS = 2048
_NUM_COLS = 2048
_TOTAL = _NUM_ROWS * _NUM_COLS
_TABLE = 16384
_NW = 32
_PER_W = _TOTAL // _NW
_CHUNK = 16384
_NCHUNK = _PER_W // _CHUNK
_L = 16


def _make_sc_gather():
    mesh = plsc.VectorSubcoreMesh(core_axis_name="c", subcore_axis_name="s")

    @functools.partial(
        pl.kernel,
        mesh=mesh,
        out_type=jax.ShapeDtypeStruct((_TOTAL,), jnp.float32),
        scratch_types=[
            pltpu.VMEM((_TABLE,), jnp.float32),
            pltpu.VMEM((_CHUNK,), jnp.int32),
            pltpu.VMEM((_CHUNK,), jnp.float32),
        ],
        compiler_params=pltpu.CompilerParams(needs_layout_passes=False),
    )
    def k(x_hbm, idx_hbm, out_hbm, table_v, idx_v, out_v):
        wid = lax.axis_index("s") * 2 + lax.axis_index("c")
        base = wid * _PER_W
        pltpu.sync_copy(x_hbm, table_v)

        def chunk_body(ci, carry):
            off = base + ci * _CHUNK
            pltpu.sync_copy(idx_hbm.at[pl.ds(off, _CHUNK)], idx_v)

            def gather_body(i, c):
                iv = idx_v[pl.ds(i * _L, _L)]
                out_v[pl.ds(i * _L, _L)] = plsc.load_gather(table_v, [iv])
                return c

            lax.fori_loop(0, _CHUNK // _L, gather_body, 0, unroll=8)
            pltpu.sync_copy(out_v, out_hbm.at[pl.ds(off, _CHUNK)])
            return carry

        lax.fori_loop(0, _NCHUNK, chunk_body, 0)

    return k


_sc_gather = _make_sc_gather()


def kernel(X, idx_weight):
    idx_flat = idx_weight.astype(jnp.int32).reshape(_TOTAL)
    out = _sc_gather(X.astype(jnp.float32), idx_flat)
    return out.reshape(_NUM_ROWS, _NUM_COLS)


# speedup vs baseline: 932.5511x; 3.1874x over previous
import functools

import jax
import jax.numpy as jnp
from jax import lax
from jax.experimental import pallas as pl
from jax.experimental.pallas import tpu as pltpu
from jax.experimental.pallas import tpu_sc as plsc

_NUM_ROWS = 2048
_NUM_COLS = 2048
_TABLE = 16384
_NW = 32
_ROWS_PER_W = _NUM_ROWS // _NW
_CHUNK_ROWS = 8
_NCHUNK = _ROWS_PER_W // _CHUNK_ROWS
_L = 16
_GROUPS = _NUM_COLS // _L


def _make_sc_gather():
    mesh = plsc.VectorSubcoreMesh(core_axis_name="c", subcore_axis_name="s")

    @functools.partial(
        pl.kernel,
        mesh=mesh,
        out_type=jax.ShapeDtypeStruct((_NUM_ROWS, _NUM_COLS), jnp.float32),
        scratch_types=[
            pltpu.VMEM((_TABLE,), jnp.float32),
            pltpu.VMEM((2, _CHUNK_ROWS, _NUM_COLS), jnp.int32),
            pltpu.VMEM((2, _CHUNK_ROWS, _NUM_COLS), jnp.float32),
            pltpu.SemaphoreType.DMA((2,)),
            pltpu.SemaphoreType.DMA((2,)),
        ],
        compiler_params=pltpu.CompilerParams(needs_layout_passes=False),
    )
    def k(x_hbm, idx_hbm, out_hbm, table_v, idx_v, out_v, sem_in, sem_out):
        wid = lax.axis_index("s") * 2 + lax.axis_index("c")
        row0 = wid * _ROWS_PER_W
        pltpu.sync_copy(x_hbm, table_v)

        def in_copy(ci, slot):
            return pltpu.make_async_copy(
                idx_hbm.at[pl.ds(row0 + ci * _CHUNK_ROWS, _CHUNK_ROWS), :],
                idx_v.at[slot],
                sem_in.at[slot],
            )

        def out_copy(ci, slot):
            return pltpu.make_async_copy(
                out_v.at[slot],
                out_hbm.at[pl.ds(row0 + ci * _CHUNK_ROWS, _CHUNK_ROWS), :],
                sem_out.at[slot],
            )

        in_copy(0, 0).start()
        for ci in range(_NCHUNK):
            slot = ci & 1
            in_copy(ci, slot).wait()
            if ci + 1 < _NCHUNK:
                in_copy(ci + 1, slot ^ 1).start()
            if ci >= 2:
                out_copy(ci - 2, slot).wait()
            for r in range(_CHUNK_ROWS):
                @plsc.parallel_loop(0, _GROUPS, unroll=8)
                def _(i, slot=slot, r=r):
                    iv = idx_v[slot, r, pl.ds(i * _L, _L)]
                    out_v[slot, r, pl.ds(i * _L, _L)] = plsc.load_gather(
                        table_v, [iv]
                    )
            out_copy(ci, slot).start()
        out_copy(_NCHUNK - 2, 0).wait()
        out_copy(_NCHUNK - 1, 1).wait()

    return k


_sc_gather = _make_sc_gather()


def kernel(X, idx_weight):
    return _sc_gather(
        X.astype(jnp.float32), idx_weight.astype(jnp.int32)
    )
